# Initial kernel scaffold; baseline (speedup 1.0000x reference)
import jax
import jax.numpy as jnp
from jax.experimental import pallas as pl


def kernel(input, index, _):
    raise NotImplementedError("write your pallas kernel here")


# speedup vs baseline: 1.4377x; 1.4377x over previous
import functools

import jax
import jax.numpy as jnp
from jax import lax
from jax.experimental import pallas as pl
from jax.experimental.pallas import tpu as pltpu
from jax.experimental.pallas import tpu_sc as plsc

N_IN = 160000
N_FEAT = 256
N_OUT = 10000

L = 16
N_CORES = 2
N_SUB = 16
N_TILES = N_CORES * N_SUB

NOWN = 312
NOWN_LAST = N_OUT - (N_TILES - 1) * NOWN
ACC_ROWS = 336
TRASH = NOWN_LAST + 1

BLOCK = 64
SBS = 8192
NSB = -(-N_IN // SBS)
IDX_PAD = NSB * SBS
SENTINEL = 4 * N_OUT
SUBBLOCKS = SBS // BLOCK

_COLS = None


def _segment_sum_sc(inp, idx_pad, zeros):
    mesh = plsc.VectorSubcoreMesh(core_axis_name="c", subcore_axis_name="s")

    @functools.partial(
        pl.kernel,
        mesh=mesh,
        compiler_params=pltpu.CompilerParams(needs_layout_passes=False),
        out_type=jax.ShapeDtypeStruct((N_OUT, N_FEAT), jnp.float32),
        scratch_types=[
            pltpu.VMEM((SBS,), jnp.int32),
            pltpu.VMEM((BLOCK, N_FEAT), jnp.float32),
            pltpu.VMEM((ACC_ROWS, N_FEAT), jnp.float32),
        ],
    )
    def k(inp_hbm, idx_hbm, zeros_hbm, out_hbm, idx_v, rows_v, acc_v):
        c = lax.axis_index("c")
        s = lax.axis_index("s")
        w = c * N_SUB + s
        lo = w * NOWN
        bound = jnp.where(w == N_TILES - 1, NOWN_LAST, NOWN)
        hi = lo + bound

        cols = [lax.iota(jnp.int32, L) + j * L for j in range(N_FEAT // L)]

        pltpu.sync_copy(zeros_hbm, acc_v)

        def sb_body(sb, carry):
            pltpu.sync_copy(idx_hbm.at[pl.ds(sb * SBS, SBS)], idx_v)
            sb_min = jnp.min(idx_v[pl.ds(0, L)])
            sb_max = jnp.max(idx_v[pl.ds(SBS - L, L)])

            @pl.when((sb_max >= lo) & (sb_min < hi))
            def _():
                def b_body(b, carry2):
                    r0 = b * BLOCK
                    b_min = jnp.min(idx_v[pl.ds(r0, L)])
                    b_max = jnp.max(idx_v[pl.ds(r0 + BLOCK - L, L)])

                    @pl.when((b_max >= lo) & (b_min < hi))
                    def _():
                        pltpu.sync_copy(
                            inp_hbm.at[pl.ds(sb * SBS + r0, BLOCK)], rows_v)
                        for g in range(BLOCK // L):
                            lvec = idx_v[pl.ds(r0 + g * L, L)] - lo
                            ok = (lvec >= 0) & (lvec < bound)
                            lv = jnp.where(ok, lvec, TRASH)
                            for r in range(L):
                                row = lax.gather(
                                    lv,
                                    jnp.full((L, 1), r, jnp.int32),
                                    lax.GatherDimensionNumbers(
                                        offset_dims=(),
                                        collapsed_slice_dims=(0,),
                                        start_index_map=(0,)),
                                    (1,),
                                    mode=lax.GatherScatterMode.PROMISE_IN_BOUNDS)
                                for j in range(N_FEAT // L):
                                    plsc.addupdate_scatter(
                                        acc_v, [row, cols[j]],
                                        rows_v[g * L + r, pl.ds(j * L, L)])

                    return carry2

                lax.fori_loop(0, SUBBLOCKS, b_body, 0)

            return carry

        lax.fori_loop(0, NSB, sb_body, 0)

        @pl.when(w < N_TILES - 1)
        def _():
            pltpu.sync_copy(acc_v.at[pl.ds(0, NOWN)],
                            out_hbm.at[pl.ds(lo, NOWN)])

        @pl.when(w == N_TILES - 1)
        def _():
            pltpu.sync_copy(acc_v.at[pl.ds(0, NOWN_LAST)],
                            out_hbm.at[pl.ds(lo, NOWN_LAST)])

    return k(inp, idx_pad, zeros)


def kernel(input, index, _):
    idx_pad = jnp.concatenate(
        [index, jnp.full((IDX_PAD - N_IN,), SENTINEL, jnp.int32)])
    zeros = jnp.zeros((ACC_ROWS, N_FEAT), jnp.float32)
    out = _segment_sum_sc(input, idx_pad, zeros)
    return (input, index, out)


# speedup vs baseline: 1.7503x; 1.2175x over previous
import functools

import jax
import jax.numpy as jnp
from jax import lax
from jax.experimental import pallas as pl
from jax.experimental.pallas import tpu as pltpu
from jax.experimental.pallas import tpu_sc as plsc

N_IN = 160000
N_FEAT = 256
N_OUT = 10000

L = 16
N_CORES = 2
N_SUB = 16
N_TILES = N_CORES * N_SUB

NOWN = 312
NOWN_LAST = N_OUT - (N_TILES - 1) * NOWN
ACC_ROWS = 336
TRASH = NOWN_LAST + 1

BLOCK = 64
SBS = 8192
NSB = -(-N_IN // SBS)
IDX_PAD = NSB * SBS
SENTINEL = 4 * N_OUT
SUBBLOCKS = SBS // BLOCK

_COLS = None


def _segment_sum_sc(inp, idx_pad, zeros):
    mesh = plsc.VectorSubcoreMesh(core_axis_name="c", subcore_axis_name="s")

    @functools.partial(
        pl.kernel,
        mesh=mesh,
        compiler_params=pltpu.CompilerParams(needs_layout_passes=False),
        out_type=jax.ShapeDtypeStruct((N_OUT, N_FEAT), jnp.float32),
        scratch_types=[
            pltpu.VMEM((SBS,), jnp.int32),
            pltpu.VMEM((BLOCK, N_FEAT), jnp.float32),
            pltpu.VMEM((ACC_ROWS, N_FEAT), jnp.float32),
        ],
    )
    def k(inp_hbm, idx_hbm, zeros_hbm, out_hbm, idx_v, rows_v, acc_v):
        c = lax.axis_index("c")
        s = lax.axis_index("s")
        w = c * N_SUB + s
        lo = w * NOWN
        bound = jnp.where(w == N_TILES - 1, NOWN_LAST, NOWN)
        hi = lo + bound

        cols = [lax.iota(jnp.int32, L) + j * L for j in range(N_FEAT // L)]

        pltpu.sync_copy(zeros_hbm, acc_v)

        def sb_body(sb, carry):
            pltpu.sync_copy(idx_hbm.at[pl.ds(sb * SBS, SBS)], idx_v)
            sb_min = jnp.min(idx_v[pl.ds(0, L)])
            sb_max = jnp.max(idx_v[pl.ds(SBS - L, L)])

            @pl.when((sb_max >= lo) & (sb_min < hi))
            def _():
                def b_body(b, carry2):
                    r0 = b * BLOCK
                    b_min = jnp.min(idx_v[pl.ds(r0, L)])
                    b_max = jnp.max(idx_v[pl.ds(r0 + BLOCK - L, L)])

                    @pl.when((b_max >= lo) & (b_min < hi))
                    def _():
                        pltpu.sync_copy(
                            inp_hbm.at[pl.ds(sb * SBS + r0, BLOCK)], rows_v)
                        for g in range(BLOCK // L):
                            lvec = idx_v[pl.ds(r0 + g * L, L)] - lo
                            ok = (lvec >= 0) & (lvec < bound)
                            lv = jnp.where(ok, lvec, TRASH)
                            for r in range(L):
                                vals = [rows_v[g * L + r, pl.ds(j * L, L)]
                                        for j in range(N_FEAT // L)]
                                row = lax.gather(
                                    lv,
                                    jnp.full((L, 1), r, jnp.int32),
                                    lax.GatherDimensionNumbers(
                                        offset_dims=(),
                                        collapsed_slice_dims=(0,),
                                        start_index_map=(0,)),
                                    (1,),
                                    mode=lax.GatherScatterMode.PROMISE_IN_BOUNDS)
                                for j in range(N_FEAT // L):
                                    plsc.addupdate_scatter(
                                        acc_v, [row, cols[j]], vals[j])

                    return carry2

                lax.fori_loop(0, SUBBLOCKS, b_body, 0)

            return carry

        lax.fori_loop(0, NSB, sb_body, 0)

        @pl.when(w < N_TILES - 1)
        def _():
            pltpu.sync_copy(acc_v.at[pl.ds(0, NOWN)],
                            out_hbm.at[pl.ds(lo, NOWN)])

        @pl.when(w == N_TILES - 1)
        def _():
            pltpu.sync_copy(acc_v.at[pl.ds(0, NOWN_LAST)],
                            out_hbm.at[pl.ds(lo, NOWN_LAST)])

    return k(inp, idx_pad, zeros)


def kernel(input, index, _):
    idx_pad = jnp.concatenate(
        [index, jnp.full((IDX_PAD - N_IN,), SENTINEL, jnp.int32)])
    zeros = jnp.zeros((ACC_ROWS, N_FEAT), jnp.float32)
    out = _segment_sum_sc(input, idx_pad, zeros)
    return (input, index, out)
